# baseline (speedup 1.0000x reference)
import functools

import jax
import jax.numpy as jnp
from jax import lax
from jax.experimental import pallas as pl
from jax.experimental.pallas import tpu as pltpu
from jax.experimental.pallas import tpu_sc as plsc

_N = 1024 * 1024
_NC = 2
_NS = 16
_NW = _NC * _NS
_CHUNK = _N // _NW
_NB = 2
_SUB = _CHUNK // _NB
_L = 16


def _sc_body(w_hbm, n_hbm, o_hbm,
             w0, w1, n0, n1, o0, o1,
             sw0, sw1, sn0, sn1, so0, so1):
    wid = lax.axis_index("s") * _NC + lax.axis_index("c")
    base = wid * _CHUNK
    wv = (w0, w1)
    nv = (n0, n1)
    ov = (o0, o1)
    sw = (sw0, sw1)
    sn = (sn0, sn1)
    so = (so0, so1)

    def start_in(g):
        b = g & 1
        off = base + g * _SUB
        hw = pltpu.async_copy(w_hbm.at[pl.ds(off, _SUB)], wv[b], sw[b])
        hn = pltpu.async_copy(n_hbm.at[pl.ds(off, _SUB)], nv[b], sn[b])
        return hw, hn

    in_h = [None, None]
    out_h = [None, None]
    in_h[0] = start_in(0)
    for g in range(_NB):
        b = g & 1
        if g + 1 < _NB:
            in_h[1 - b] = start_in(g + 1)
        hw, hn = in_h[b]
        hw.wait()
        hn.wait()
        if out_h[b] is not None:
            out_h[b].wait()
        wb, nb, ob = wv[b], nv[b], ov[b]

        @plsc.parallel_loop(0, _SUB, step=_L, unroll=8)
        def _loop(i):
            x = (nb[pl.ds(i, _L)] - wb[pl.ds(i, _L)]) * 10.0
            ob[pl.ds(i, _L)] = 1.0 / (1.0 + jnp.exp(x))

        out_h[b] = pltpu.async_copy(
            ov[b], o_hbm.at[pl.ds(base + g * _SUB, _SUB)], so[b])
    out_h[0].wait()
    out_h[1].wait()


_sc_kernel = functools.partial(
    pl.kernel,
    mesh=plsc.VectorSubcoreMesh(core_axis_name="c", subcore_axis_name="s"),
    out_type=jax.ShapeDtypeStruct((_N,), jnp.float32),
    scratch_types=(
        [pltpu.VMEM((_SUB,), jnp.float32) for _ in range(6)]
        + [pltpu.SemaphoreType.DMA for _ in range(6)]
    ),
)(_sc_body)


def kernel(weights, noises):
    return _sc_kernel(weights, noises)


# speedup vs baseline: 5.3497x; 5.3497x over previous
import jax
import jax.numpy as jnp
from jax.experimental import pallas as pl
from jax.experimental.pallas import tpu as pltpu

_N = 1024 * 1024
_U = _N // 32
_CHUNKS = [12 * _U, 10 * _U, 6 * _U, 2 * _U, _U, _U]
_NCH = len(_CHUNKS)
_OFFS = [sum(_CHUNKS[:g]) for g in range(_NCH)]


def _body(w_hbm, n_hbm, o_hbm, *scr):
    wv = scr[0:_NCH]
    nv = scr[_NCH:2 * _NCH]
    ov = scr[2 * _NCH:3 * _NCH]
    sw = scr[3 * _NCH:4 * _NCH]
    sn = scr[4 * _NCH:5 * _NCH]
    so = scr[5 * _NCH:6 * _NCH]

    h_in = []
    for g in range(_NCH):
        hw = pltpu.make_async_copy(
            w_hbm.at[pl.ds(_OFFS[g], _CHUNKS[g])], wv[g], sw[g])
        hn = pltpu.make_async_copy(
            n_hbm.at[pl.ds(_OFFS[g], _CHUNKS[g])], nv[g], sn[g])
        hw.start()
        hn.start()
        h_in.append((hw, hn))

    h_out = []
    for g in range(_NCH):
        hw, hn = h_in[g]
        hw.wait()
        hn.wait()
        ov[g][...] = jax.nn.sigmoid((wv[g][...] - nv[g][...]) * 10.0)
        ho = pltpu.make_async_copy(
            ov[g], o_hbm.at[pl.ds(_OFFS[g], _CHUNKS[g])], so[g])
        ho.start()
        h_out.append(ho)
    for ho in h_out:
        ho.wait()


def kernel(weights, noises):
    return pl.pallas_call(
        _body,
        out_shape=jax.ShapeDtypeStruct((_N,), jnp.float32),
        in_specs=[
            pl.BlockSpec(memory_space=pl.ANY),
            pl.BlockSpec(memory_space=pl.ANY),
        ],
        out_specs=pl.BlockSpec(memory_space=pl.ANY),
        scratch_shapes=(
            [pltpu.VMEM((c,), jnp.float32) for c in _CHUNKS] * 3
            + [pltpu.SemaphoreType.DMA for _ in range(3 * _NCH)]
        ),
    )(weights, noises)
